# baseline (speedup 1.0000x reference)
import functools

import jax
import jax.numpy as jnp
from jax import lax
from jax.experimental import pallas as pl
from jax.experimental.pallas import tpu as pltpu
from jax.experimental.pallas import tpu_sc as plsc

F32 = jnp.float32
NC, NS = 2, 16
NW = NC * NS
EB = 128
DW = 16
BM = 2000


def _sc_mesh():
    return plsc.VectorSubcoreMesh(
        core_axis_name="c", subcore_axis_name="s", num_cores=NC, num_subcores=NS)


def _make_propagate(n_pad, nb, c):
    rpt = n_pad // NS
    KB = 2
    assert nb % KB == 0

    @functools.partial(
        pl.kernel,
        out_type=(jax.ShapeDtypeStruct((n_pad, c), F32),
                  jax.ShapeDtypeStruct((n_pad, c), F32)),
        mesh=_sc_mesh(),
        compiler_params=pltpu.CompilerParams(use_tc_tiling_on_sc=False),
        scratch_types=[
            pltpu.VMEM((nb, EB), jnp.int32),
            pltpu.VMEM((nb, EB), jnp.int32),
            [pltpu.VMEM((EB, c), F32) for _ in range(KB)],
            pltpu.VMEM_SHARED((n_pad, c), F32),
            pltpu.SemaphoreType.DMA,
        ],
    )
    def prop(src_hbm, dst_hbm, y_hbm, zeros_hbm, za_hbm, zb_hbm,
             srcv, dstv, gbufs, zsh, gsem):
        cid = lax.axis_index("c")
        sid = lax.axis_index("s")
        wid = sid * NC + cid
        pltpu.sync_copy(src_hbm.at[wid], srcv)
        pltpu.sync_copy(dst_hbm.at[wid], dstv)
        row0 = sid * rpt
        pltpu.sync_copy(zeros_hbm.at[pl.ds(row0, rpt)],
                        zsh.at[pl.ds(row0, rpt)])
        plsc.subcore_barrier()

        for b in range(KB):
            pltpu.async_copy(y_hbm.at[srcv.at[b]], gbufs[b], gsem)

        def body(g, carry):
            for b in range(KB):
                jt = g * KB + b
                pltpu.make_async_copy(y_hbm.at[srcv.at[jt]], gbufs[b],
                                      gsem).wait()
                pltpu.sync_copy(gbufs[b], zsh.at[dstv.at[jt]], add=True)

                @pl.when(jt + KB < nb)
                def _():
                    pltpu.async_copy(y_hbm.at[srcv.at[jt + KB]], gbufs[b],
                                     gsem)
            return carry

        lax.fori_loop(0, nb // KB, body, 0)
        plsc.subcore_barrier()

        @pl.when(cid == 0)
        def _():
            pltpu.sync_copy(zsh.at[pl.ds(row0, rpt)],
                            za_hbm.at[pl.ds(row0, rpt)])

        @pl.when(cid == 1)
        def _():
            pltpu.sync_copy(zsh.at[pl.ds(row0, rpt)],
                            zb_hbm.at[pl.ds(row0, rpt)])

    return prop


def _make_degree(n_pad, nb):
    rpt = n_pad // NS

    @functools.partial(
        pl.kernel,
        out_type=(jax.ShapeDtypeStruct((n_pad, DW), F32),
                  jax.ShapeDtypeStruct((n_pad, DW), F32)),
        mesh=_sc_mesh(),
        compiler_params=pltpu.CompilerParams(use_tc_tiling_on_sc=False),
        scratch_types=[
            pltpu.VMEM((nb, EB), jnp.int32),
            pltpu.VMEM((EB, DW), F32),
            pltpu.VMEM_SHARED((n_pad, DW), F32),
        ],
    )
    def degk(dst_hbm, ones_hbm, zeros_hbm, da_hbm, db_hbm, dstv, onesv, dsh):
        cid = lax.axis_index("c")
        sid = lax.axis_index("s")
        wid = sid * NC + cid
        pltpu.sync_copy(dst_hbm.at[wid], dstv)
        pltpu.sync_copy(ones_hbm, onesv)
        row0 = sid * rpt
        pltpu.sync_copy(zeros_hbm.at[pl.ds(row0, rpt)],
                        dsh.at[pl.ds(row0, rpt)])
        plsc.subcore_barrier()

        def body(j, carry):
            pltpu.sync_copy(onesv, dsh.at[dstv.at[j]], add=True)
            return carry

        lax.fori_loop(0, nb, body, 0)
        plsc.subcore_barrier()

        @pl.when(cid == 0)
        def _():
            pltpu.sync_copy(dsh.at[pl.ds(row0, rpt)],
                            da_hbm.at[pl.ds(row0, rpt)])

        @pl.when(cid == 1)
        def _():
            pltpu.sync_copy(dsh.at[pl.ds(row0, rpt)],
                            db_hbm.at[pl.ds(row0, rpt)])

    return degk


def _full(shape):
    return pl.BlockSpec(shape, lambda i: tuple(0 for _ in shape))


def _rows(c):
    return pl.BlockSpec((BM, c), lambda i: (i, 0))


def _rows_flat(c):
    return pl.BlockSpec((BM * c // 128, 128), lambda i: (i, 0))


def _inproj(x, dega, degb, win, bin_, w1, n):
    grid = (n // BM,)
    c0 = win.shape[1]
    c1 = w1.shape[1]

    def body(x_ref, da_ref, db_ref, w_ref, b_ref, w1_ref,
             h0_ref, dinv_ref, y1_ref):
        deg = da_ref[:, 0:1] + db_ref[:, 0:1] + 1.0
        dv = lax.rsqrt(deg)
        h0 = jnp.dot(x_ref[...], w_ref[...],
                     preferred_element_type=F32) + b_ref[...]
        h0_ref[...] = h0
        dinv_ref[...] = jnp.broadcast_to(dv, dinv_ref.shape)
        y1_ref[...] = dv * jnp.dot(h0, w1_ref[...], preferred_element_type=F32)

    return pl.pallas_call(
        body,
        grid=grid,
        in_specs=[_rows(x.shape[1]), _rows(DW), _rows(DW),
                  _full(win.shape), _full(bin_.shape), _full(w1.shape)],
        out_specs=(_rows(c0), _rows(8), _rows(c1)),
        out_shape=(jax.ShapeDtypeStruct((n, c0), F32),
                   jax.ShapeDtypeStruct((n, 8), F32),
                   jax.ShapeDtypeStruct((n, c1), F32)),
    )(x, dega, degb, win, bin_, w1)


def _matmul_main(feats, scshs, w, widths, n):
    grid = (n // BM,)
    nf = len(feats)
    c_out = w.shape[1]

    def body(*refs):
        f_refs = refs[0:nf]
        s_refs = refs[nf:2 * nf - 1]
        w_ref = refs[2 * nf - 1]
        y_ref = refs[2 * nf]
        off = widths[0]
        acc = jnp.dot(f_refs[0][...], w_ref[0:off, :],
                      preferred_element_type=F32)
        for j in range(1, nf):
            sc = s_refs[j - 1]
            o = jnp.maximum(f_refs[j][...] * sc[0:1, :] + sc[1:2, :], 0.0)
            acc = acc + jnp.dot(o, w_ref[off:off + widths[j], :],
                                preferred_element_type=F32)
            off += widths[j]
        y_ref[...] = acc

    in_specs = ([_rows(f.shape[1]) for f in feats]
                + [_full(s.shape) for s in scshs]
                + [_full(w.shape)])
    return pl.pallas_call(
        body,
        grid=grid,
        in_specs=in_specs,
        out_specs=_rows(c_out),
        out_shape=jax.ShapeDtypeStruct((n, c_out), F32),
    )(*feats, *scshs, w)


def _flat_ok(c):
    return c == 128


def _fin_tail(za, zb, y, dinv, gamma, beta, acc_next, w, off, wd, c, n):
    grid = (2, n // BM)
    c_out = w.shape[1]

    def body(za_ref, zb_ref, y_ref, dinv_ref, g_ref, b_ref, a_ref, w_ref,
             u_ref, sc_ref, y2_ref, acc):
        p = pl.program_id(0)
        i = pl.program_id(1)

        @pl.when(jnp.logical_and(p == 0, i == 0))
        def _():
            acc[...] = jnp.zeros_like(acc)

        dv = dinv_ref[:, 0:1]
        u = dv * (za_ref[...] + zb_ref[...] + y_ref[...])
        u_ref[...] = u

        @pl.when(p == 0)
        def _():
            acc[0:1, :] += jnp.sum(u, axis=0, keepdims=True)
            acc[1:2, :] += jnp.sum(u * u, axis=0, keepdims=True)

            @pl.when(i == grid[1] - 1)
            def _():
                mean = acc[0:1, :] / float(n)
                var = acc[1:2, :] / float(n) - mean * mean
                rstd = lax.rsqrt(var + 1e-5)
                scale = g_ref[...] * rstd
                shift = b_ref[...] - mean * scale
                acc[2:3, :] = scale
                acc[3:4, :] = shift
                sc_ref[...] = jnp.concatenate(
                    [scale, shift, jnp.zeros((6, c), F32)], axis=0)

        @pl.when(p == 1)
        def _():
            o = jnp.maximum(u * acc[2:3, :] + acc[3:4, :], 0.0)
            y2_ref[...] = dv * (
                a_ref[...] + jnp.dot(o, w_ref[off:off + wd, :],
                                     preferred_element_type=F32))

    zspec = pl.BlockSpec((BM, c), lambda p, i: (i, 0))
    return pl.pallas_call(
        body,
        grid=grid,
        in_specs=[zspec, zspec, zspec,
                  pl.BlockSpec((BM, 8), lambda p, i: (i, 0)),
                  pl.BlockSpec((1, c), lambda p, i: (0, 0)),
                  pl.BlockSpec((1, c), lambda p, i: (0, 0)),
                  pl.BlockSpec((BM, c_out), lambda p, i: (i * p, 0)),
                  pl.BlockSpec(w.shape, lambda p, i: (0, 0))],
        out_specs=(zspec,
                   pl.BlockSpec((8, c), lambda p, i: (0, 0)),
                   pl.BlockSpec((BM, c_out), lambda p, i: (i * p, 0))),
        out_shape=(jax.ShapeDtypeStruct((n, c), F32),
                   jax.ShapeDtypeStruct((8, c), F32),
                   jax.ShapeDtypeStruct((n, c_out), F32)),
        scratch_shapes=[pltpu.VMEM((8, c), F32)],
    )(za, zb, y, dinv, gamma, beta, acc_next, w)


def _fin(za, zb, y, dinv, gamma, beta, c, n):
    grid = (n // BM,)
    flat = _flat_ok(c)

    def body(za_ref, zb_ref, y_ref, dinv_ref, g_ref, b_ref, u_ref, sc_ref, acc):
        i = pl.program_id(0)

        @pl.when(i == 0)
        def _():
            acc[...] = jnp.zeros_like(acc)

        dv = dinv_ref[:, 0:1]
        z = za_ref[...] + zb_ref[...] + y_ref[...]
        u = dv * (z.reshape(BM, c) if flat else z)
        u_ref[...] = u
        acc[0:1, :] += jnp.sum(u, axis=0, keepdims=True)
        acc[1:2, :] += jnp.sum(u * u, axis=0, keepdims=True)

        @pl.when(i == grid[0] - 1)
        def _():
            mean = acc[0:1, :] / float(n)
            var = acc[1:2, :] / float(n) - mean * mean
            rstd = lax.rsqrt(var + 1e-5)
            scale = g_ref[...] * rstd
            shift = b_ref[...] - mean * scale
            sc_ref[...] = jnp.concatenate(
                [scale, shift, jnp.zeros((6, c), F32)], axis=0)

    zspec = _rows_flat(c) if flat else _rows(c)
    return pl.pallas_call(
        body,
        grid=grid,
        in_specs=[zspec, zspec, zspec, _rows(8),
                  _full((1, c)), _full((1, c))],
        out_specs=(_rows(c), _full((8, c))),
        out_shape=(jax.ShapeDtypeStruct((n, c), F32),
                   jax.ShapeDtypeStruct((8, c), F32)),
        scratch_shapes=[pltpu.VMEM((8, c), F32)],
    )(za, zb, y, dinv, gamma, beta)


def _pool_cls(u, sc, batch2, w1, b1, w2, b2, n, g):
    grid = (n // BM,)
    c = u.shape[1]
    nclass = w2.shape[1]

    def body(u_ref, sc_ref, b_ref, w1_ref, b1_ref, w2_ref, b2_ref,
             out_ref, accg, accc):
        i = pl.program_id(0)

        @pl.when(i == 0)
        def _():
            accg[...] = jnp.zeros_like(accg)
            accc[...] = jnp.zeros_like(accc)

        scale = sc_ref[0:1, :]
        shift = sc_ref[1:2, :]
        o = jnp.maximum(u_ref[...] * scale + shift, 0.0)
        gid = b_ref[...]
        onehot = (gid == lax.broadcasted_iota(jnp.int32, (1, g), 1)).astype(F32)
        dn = (((0,), (0,)), ((), ()))
        accg[...] += lax.dot_general(onehot, o, dn, preferred_element_type=F32)
        accc[...] += lax.dot_general(onehot, jnp.ones_like(o), dn,
                                     preferred_element_type=F32)

        @pl.when(i == grid[0] - 1)
        def _():
            gm = accg[...] / jnp.maximum(accc[...], 1.0)
            z1 = jnp.maximum(
                jnp.dot(gm, w1_ref[...], preferred_element_type=F32)
                + b1_ref[...], 0.0)
            out_ref[...] = (jnp.dot(z1, w2_ref[...], preferred_element_type=F32)
                            + b2_ref[...])

    return pl.pallas_call(
        body,
        grid=grid,
        in_specs=[_rows(c), _full((8, c)), _rows(1),
                  _full(w1.shape), _full(b1.shape),
                  _full(w2.shape), _full(b2.shape)],
        out_specs=_full((g, nclass)),
        out_shape=jax.ShapeDtypeStruct((g, nclass), F32),
        scratch_shapes=[pltpu.VMEM((g, c), F32), pltpu.VMEM((g, c), F32)],
    )(u, sc, batch2, w1, b1, w2, b2)


def kernel(x, edge_index, batch, params):
    n = x.shape[0]
    e = edge_index.shape[1]
    n_pad = -(-(n + DW) // 128) * 128

    src = edge_index[0].astype(jnp.int32)
    dst = edge_index[1].astype(jnp.int32)

    ep = -(-e // (NW * EB)) * (NW * EB)
    extra = ep - e
    padv = jnp.arange(extra, dtype=jnp.int32) % DW
    src3 = jnp.concatenate([src, padv]).reshape(NW, -1, EB)
    dst3 = jnp.concatenate([dst, n + padv]).reshape(NW, -1, EB)
    nb = src3.shape[1]

    zeros16 = jnp.zeros((n_pad, DW), F32)
    ones16 = jnp.ones((EB, DW), F32)

    dega, degb = _make_degree(n_pad, nb)(dst3, ones16, zeros16)

    convs = []
    for blk in params["blocks"]:
        for lp in blk:
            convs.append((lp["gcn"]["W"], lp["bn"]["gamma"], lp["bn"]["beta"]))
    convs.append((params["final_gcn"]["W"], params["final_bn"]["gamma"],
                  params["final_bn"]["beta"]))

    win = params["in_proj"]["W"]
    bin_ = params["in_proj"]["b"].reshape(1, -1)
    h0, dinv, y = _inproj(x, dega, degb, win, bin_, convs[0][0], n)

    feats = [h0]
    scshs = []
    widths = [h0.shape[1]]
    zeros_cache = {}
    for k, (w, gamma, beta) in enumerate(convs):
        c_out = w.shape[1]
        if c_out not in zeros_cache:
            zeros_cache[c_out] = jnp.zeros((n_pad, c_out), F32)
        za, zb = _make_propagate(n_pad, nb, c_out)(
            src3, dst3, y, zeros_cache[c_out])
        if k + 1 < len(convs):
            w_next = convs[k + 1][0]
            acc_next = _matmul_main(feats, scshs, w_next, widths, n)
            u, scsh, y = _fin_tail(za, zb, y, dinv, gamma.reshape(1, -1),
                                   beta.reshape(1, -1), acc_next, w_next,
                                   sum(widths), c_out, c_out, n)
        else:
            u, scsh = _fin(za, zb, y, dinv, gamma.reshape(1, -1),
                           beta.reshape(1, -1), c_out, n)
        feats.append(u)
        scshs.append(scsh)
        widths.append(c_out)

    u_final = feats.pop()
    sc_final = scshs.pop()
    batch2 = batch.astype(jnp.int32).reshape(n, 1)
    g = 64
    return _pool_cls(u_final, sc_final, batch2,
                     params["cls1"]["W"], params["cls1"]["b"].reshape(1, -1),
                     params["cls2"]["W"], params["cls2"]["b"].reshape(1, -1),
                     n, g)


# speedup vs baseline: 1.0316x; 1.0316x over previous
import functools

import jax
import jax.numpy as jnp
from jax import lax
from jax.experimental import pallas as pl
from jax.experimental.pallas import tpu as pltpu
from jax.experimental.pallas import tpu_sc as plsc

F32 = jnp.float32
NC, NS = 2, 16
NW = NC * NS
EB = 128
DW = 16
BM = 2000


def _sc_mesh():
    return plsc.VectorSubcoreMesh(
        core_axis_name="c", subcore_axis_name="s", num_cores=NC, num_subcores=NS)


def _make_propagate(n_pad, nb, c):
    rpt = n_pad // NS
    KB = 2
    assert nb % KB == 0

    @functools.partial(
        pl.kernel,
        out_type=(jax.ShapeDtypeStruct((n_pad, c), F32),
                  jax.ShapeDtypeStruct((n_pad, c), F32)),
        mesh=_sc_mesh(),
        compiler_params=pltpu.CompilerParams(use_tc_tiling_on_sc=False),
        scratch_types=[
            pltpu.VMEM((nb, EB), jnp.int32),
            pltpu.VMEM((nb, EB), jnp.int32),
            [pltpu.VMEM((EB, c), F32) for _ in range(KB)],
            pltpu.VMEM_SHARED((n_pad, c), F32),
            pltpu.SemaphoreType.DMA,
        ],
    )
    def prop(src_hbm, dst_hbm, y_hbm, zeros_hbm, za_hbm, zb_hbm,
             srcv, dstv, gbufs, zsh, gsem):
        cid = lax.axis_index("c")
        sid = lax.axis_index("s")
        wid = sid * NC + cid
        pltpu.sync_copy(src_hbm.at[wid], srcv)
        pltpu.sync_copy(dst_hbm.at[wid], dstv)
        row0 = sid * rpt
        pltpu.sync_copy(zeros_hbm.at[pl.ds(row0, rpt)],
                        zsh.at[pl.ds(row0, rpt)])
        plsc.subcore_barrier()

        for b in range(KB):
            pltpu.async_copy(y_hbm.at[srcv.at[b]], gbufs[b], gsem)

        def body(g, carry):
            for b in range(KB):
                jt = g * KB + b
                pltpu.make_async_copy(y_hbm.at[srcv.at[jt]], gbufs[b],
                                      gsem).wait()
                pltpu.sync_copy(gbufs[b], zsh.at[dstv.at[jt]], add=True)

                @pl.when(jt + KB < nb)
                def _():
                    pltpu.async_copy(y_hbm.at[srcv.at[jt + KB]], gbufs[b],
                                     gsem)
            return carry

        lax.fori_loop(0, nb // KB, body, 0)
        plsc.subcore_barrier()

        @pl.when(cid == 0)
        def _():
            pltpu.sync_copy(zsh.at[pl.ds(row0, rpt)],
                            za_hbm.at[pl.ds(row0, rpt)])

        @pl.when(cid == 1)
        def _():
            pltpu.sync_copy(zsh.at[pl.ds(row0, rpt)],
                            zb_hbm.at[pl.ds(row0, rpt)])

    return prop


def _make_degree(n_pad, nb):
    rpt = n_pad // NS

    @functools.partial(
        pl.kernel,
        out_type=(jax.ShapeDtypeStruct((n_pad, DW), F32),
                  jax.ShapeDtypeStruct((n_pad, DW), F32)),
        mesh=_sc_mesh(),
        compiler_params=pltpu.CompilerParams(use_tc_tiling_on_sc=False),
        scratch_types=[
            pltpu.VMEM((nb, EB), jnp.int32),
            pltpu.VMEM((EB, DW), F32),
            pltpu.VMEM_SHARED((n_pad, DW), F32),
        ],
    )
    def degk(dst_hbm, ones_hbm, zeros_hbm, da_hbm, db_hbm, dstv, onesv, dsh):
        cid = lax.axis_index("c")
        sid = lax.axis_index("s")
        wid = sid * NC + cid
        pltpu.sync_copy(dst_hbm.at[wid], dstv)
        pltpu.sync_copy(ones_hbm, onesv)
        row0 = sid * rpt
        pltpu.sync_copy(zeros_hbm.at[pl.ds(row0, rpt)],
                        dsh.at[pl.ds(row0, rpt)])
        plsc.subcore_barrier()

        def body(j, carry):
            pltpu.sync_copy(onesv, dsh.at[dstv.at[j]], add=True)
            return carry

        lax.fori_loop(0, nb, body, 0)
        plsc.subcore_barrier()

        @pl.when(cid == 0)
        def _():
            pltpu.sync_copy(dsh.at[pl.ds(row0, rpt)],
                            da_hbm.at[pl.ds(row0, rpt)])

        @pl.when(cid == 1)
        def _():
            pltpu.sync_copy(dsh.at[pl.ds(row0, rpt)],
                            db_hbm.at[pl.ds(row0, rpt)])

    return degk


def _full(shape):
    return pl.BlockSpec(shape, lambda i: tuple(0 for _ in shape))


def _rows(c):
    return pl.BlockSpec((BM, c), lambda i: (i, 0))


def _rows_flat(c):
    return pl.BlockSpec((BM * c // 128, 128), lambda i: (i, 0))


def _inproj(x, dega, degb, win, bin_, w1, n):
    grid = (n // BM,)
    c0 = win.shape[1]
    c1 = w1.shape[1]

    def body(x_ref, da_ref, db_ref, w_ref, b_ref, w1_ref,
             h0_ref, dinv_ref, y1_ref):
        deg = da_ref[:, 0:1] + db_ref[:, 0:1] + 1.0
        dv = lax.rsqrt(deg)
        h0 = jnp.dot(x_ref[...], w_ref[...],
                     preferred_element_type=F32) + b_ref[...]
        h0_ref[...] = h0
        dinv_ref[...] = jnp.broadcast_to(dv, dinv_ref.shape)
        y1_ref[...] = dv * jnp.dot(h0, w1_ref[...], preferred_element_type=F32)

    return pl.pallas_call(
        body,
        grid=grid,
        in_specs=[_rows(x.shape[1]), _rows(DW), _rows(DW),
                  _full(win.shape), _full(bin_.shape), _full(w1.shape)],
        out_specs=(_rows(c0), _rows(8), _rows(c1)),
        out_shape=(jax.ShapeDtypeStruct((n, c0), F32),
                   jax.ShapeDtypeStruct((n, 8), F32),
                   jax.ShapeDtypeStruct((n, c1), F32)),
    )(x, dega, degb, win, bin_, w1)


def _matmul_main(feats, scshs, w, widths, n):
    grid = (n // BM,)
    nf = len(feats)
    c_out = w.shape[1]

    def body(*refs):
        f_refs = refs[0:nf]
        s_refs = refs[nf:2 * nf - 1]
        w_ref = refs[2 * nf - 1]
        y_ref = refs[2 * nf]
        off = widths[0]
        acc = jnp.dot(f_refs[0][...], w_ref[0:off, :],
                      preferred_element_type=F32)
        for j in range(1, nf):
            sc = s_refs[j - 1]
            o = jnp.maximum(f_refs[j][...] * sc[0:1, :] + sc[1:2, :], 0.0)
            acc = acc + jnp.dot(o, w_ref[off:off + widths[j], :],
                                preferred_element_type=F32)
            off += widths[j]
        y_ref[...] = acc

    in_specs = ([_rows(f.shape[1]) for f in feats]
                + [_full(s.shape) for s in scshs]
                + [_full(w.shape)])
    return pl.pallas_call(
        body,
        grid=grid,
        in_specs=in_specs,
        out_specs=_rows(c_out),
        out_shape=jax.ShapeDtypeStruct((n, c_out), F32),
    )(*feats, *scshs, w)


def _flat_ok(c):
    return c == 128


def _matmul_tail(acc, u, scsh, dinv, w, off, wd, n):
    grid = (n // BM,)
    c_out = w.shape[1]
    c = u.shape[1]

    def body(a_ref, u_ref, sc_ref, dinv_ref, w_ref, y_ref):
        o = jnp.maximum(u_ref[...] * sc_ref[0:1, :] + sc_ref[1:2, :], 0.0)
        y_ref[...] = dinv_ref[:, 0:1] * (
            a_ref[...] + jnp.dot(o, w_ref[off:off + wd, :],
                                 preferred_element_type=F32))

    return pl.pallas_call(
        body,
        grid=grid,
        in_specs=[_rows(c_out), _rows(c), _full(scsh.shape), _rows(8),
                  _full(w.shape)],
        out_specs=_rows(c_out),
        out_shape=jax.ShapeDtypeStruct((n, c_out), F32),
    )(acc, u, scsh, dinv, w)


def _fin(za, zb, y, dinv, gamma, beta, c, n):
    grid = (n // BM,)
    flat = _flat_ok(c)

    def body(za_ref, zb_ref, y_ref, dinv_ref, g_ref, b_ref, u_ref, sc_ref, acc):
        i = pl.program_id(0)

        @pl.when(i == 0)
        def _():
            acc[...] = jnp.zeros_like(acc)

        dv = dinv_ref[:, 0:1]
        z = za_ref[...] + zb_ref[...] + y_ref[...]
        u = dv * (z.reshape(BM, c) if flat else z)
        u_ref[...] = u
        acc[0:1, :] += jnp.sum(u, axis=0, keepdims=True)
        acc[1:2, :] += jnp.sum(u * u, axis=0, keepdims=True)

        @pl.when(i == grid[0] - 1)
        def _():
            mean = acc[0:1, :] / float(n)
            var = acc[1:2, :] / float(n) - mean * mean
            rstd = lax.rsqrt(var + 1e-5)
            scale = g_ref[...] * rstd
            shift = b_ref[...] - mean * scale
            sc_ref[...] = jnp.concatenate(
                [scale, shift, jnp.zeros((6, c), F32)], axis=0)

    zspec = _rows_flat(c) if flat else _rows(c)
    return pl.pallas_call(
        body,
        grid=grid,
        in_specs=[zspec, zspec, zspec, _rows(8),
                  _full((1, c)), _full((1, c))],
        out_specs=(_rows(c), _full((8, c))),
        out_shape=(jax.ShapeDtypeStruct((n, c), F32),
                   jax.ShapeDtypeStruct((8, c), F32)),
        scratch_shapes=[pltpu.VMEM((8, c), F32)],
    )(za, zb, y, dinv, gamma, beta)


def _pool_cls(u, sc, batch2, w1, b1, w2, b2, n, g):
    grid = (n // BM,)
    c = u.shape[1]
    nclass = w2.shape[1]

    def body(u_ref, sc_ref, b_ref, w1_ref, b1_ref, w2_ref, b2_ref,
             out_ref, accg, accc):
        i = pl.program_id(0)

        @pl.when(i == 0)
        def _():
            accg[...] = jnp.zeros_like(accg)
            accc[...] = jnp.zeros_like(accc)

        scale = sc_ref[0:1, :]
        shift = sc_ref[1:2, :]
        o = jnp.maximum(u_ref[...] * scale + shift, 0.0)
        gid = b_ref[...]
        onehot = (gid == lax.broadcasted_iota(jnp.int32, (1, g), 1)).astype(F32)
        dn = (((0,), (0,)), ((), ()))
        accg[...] += lax.dot_general(onehot, o, dn, preferred_element_type=F32)
        accc[...] += lax.dot_general(onehot, jnp.ones_like(o), dn,
                                     preferred_element_type=F32)

        @pl.when(i == grid[0] - 1)
        def _():
            gm = accg[...] / jnp.maximum(accc[...], 1.0)
            z1 = jnp.maximum(
                jnp.dot(gm, w1_ref[...], preferred_element_type=F32)
                + b1_ref[...], 0.0)
            out_ref[...] = (jnp.dot(z1, w2_ref[...], preferred_element_type=F32)
                            + b2_ref[...])

    return pl.pallas_call(
        body,
        grid=grid,
        in_specs=[_rows(c), _full((8, c)), _rows(1),
                  _full(w1.shape), _full(b1.shape),
                  _full(w2.shape), _full(b2.shape)],
        out_specs=_full((g, nclass)),
        out_shape=jax.ShapeDtypeStruct((g, nclass), F32),
        scratch_shapes=[pltpu.VMEM((g, c), F32), pltpu.VMEM((g, c), F32)],
    )(u, sc, batch2, w1, b1, w2, b2)


def kernel(x, edge_index, batch, params):
    n = x.shape[0]
    e = edge_index.shape[1]
    n_pad = -(-(n + DW) // 128) * 128

    src = edge_index[0].astype(jnp.int32)
    dst = edge_index[1].astype(jnp.int32)

    ep = -(-e // (NW * EB)) * (NW * EB)
    extra = ep - e
    padv = jnp.arange(extra, dtype=jnp.int32) % DW
    src3 = jnp.concatenate([src, padv]).reshape(NW, -1, EB)
    dst3 = jnp.concatenate([dst, n + padv]).reshape(NW, -1, EB)
    nb = src3.shape[1]

    zeros16 = jnp.zeros((n_pad, DW), F32)
    ones16 = jnp.ones((EB, DW), F32)

    dega, degb = _make_degree(n_pad, nb)(dst3, ones16, zeros16)

    convs = []
    for blk in params["blocks"]:
        for lp in blk:
            convs.append((lp["gcn"]["W"], lp["bn"]["gamma"], lp["bn"]["beta"]))
    convs.append((params["final_gcn"]["W"], params["final_bn"]["gamma"],
                  params["final_bn"]["beta"]))

    win = params["in_proj"]["W"]
    bin_ = params["in_proj"]["b"].reshape(1, -1)
    h0, dinv, y = _inproj(x, dega, degb, win, bin_, convs[0][0], n)

    feats = [h0]
    scshs = []
    widths = [h0.shape[1]]
    zeros_cache = {}
    for k, (w, gamma, beta) in enumerate(convs):
        c_out = w.shape[1]
        if c_out not in zeros_cache:
            zeros_cache[c_out] = jnp.zeros((n_pad, c_out), F32)
        za, zb = _make_propagate(n_pad, nb, c_out)(
            src3, dst3, y, zeros_cache[c_out])
        if k + 1 < len(convs):
            w_next = convs[k + 1][0]
            acc_next = _matmul_main(feats, scshs, w_next, widths, n)
        u, scsh = _fin(za, zb, y, dinv, gamma.reshape(1, -1),
                       beta.reshape(1, -1), c_out, n)
        feats.append(u)
        scshs.append(scsh)
        widths.append(c_out)
        if k + 1 < len(convs):
            y = _matmul_tail(acc_next, u, scsh, dinv, w_next,
                             sum(widths[:-1]), c_out, n)

    u_final = feats.pop()
    sc_final = scshs.pop()
    batch2 = batch.astype(jnp.int32).reshape(n, 1)
    g = 64
    return _pool_cls(u_final, sc_final, batch2,
                     params["cls1"]["W"], params["cls1"]["b"].reshape(1, -1),
                     params["cls2"]["W"], params["cls2"]["b"].reshape(1, -1),
                     n, g)


# speedup vs baseline: 1.0410x; 1.0091x over previous
import functools

import jax
import jax.numpy as jnp
from jax import lax
from jax.experimental import pallas as pl
from jax.experimental.pallas import tpu as pltpu
from jax.experimental.pallas import tpu_sc as plsc

F32 = jnp.float32
NC, NS = 2, 16
NW = NC * NS
EB = 128
DW = 16
BM = 2000


def _sc_mesh():
    return plsc.VectorSubcoreMesh(
        core_axis_name="c", subcore_axis_name="s", num_cores=NC, num_subcores=NS)


def _make_propagate(n_pad, nb, c):
    rpt = n_pad // NS
    KB = 2 if c >= 128 else 5
    assert nb % KB == 0

    @functools.partial(
        pl.kernel,
        out_type=(jax.ShapeDtypeStruct((n_pad, c), F32),
                  jax.ShapeDtypeStruct((n_pad, c), F32)),
        mesh=_sc_mesh(),
        compiler_params=pltpu.CompilerParams(use_tc_tiling_on_sc=False),
        scratch_types=[
            pltpu.VMEM((nb, EB), jnp.int32),
            pltpu.VMEM((nb, EB), jnp.int32),
            [pltpu.VMEM((EB, c), F32) for _ in range(KB)],
            pltpu.VMEM_SHARED((n_pad, c), F32),
            pltpu.SemaphoreType.DMA,
        ],
    )
    def prop(src_hbm, dst_hbm, y_hbm, zeros_hbm, za_hbm, zb_hbm,
             srcv, dstv, gbufs, zsh, gsem):
        cid = lax.axis_index("c")
        sid = lax.axis_index("s")
        wid = sid * NC + cid
        pltpu.sync_copy(src_hbm.at[wid], srcv)
        for b in range(KB):
            pltpu.async_copy(y_hbm.at[srcv.at[b]], gbufs[b], gsem)
        pltpu.sync_copy(dst_hbm.at[wid], dstv)
        row0 = sid * rpt
        pltpu.sync_copy(zeros_hbm.at[pl.ds(row0, rpt)],
                        zsh.at[pl.ds(row0, rpt)])
        plsc.subcore_barrier()

        def body(g, carry):
            for b in range(KB):
                jt = g * KB + b
                pltpu.make_async_copy(y_hbm.at[srcv.at[jt]], gbufs[b],
                                      gsem).wait()
                pltpu.sync_copy(gbufs[b], zsh.at[dstv.at[jt]], add=True)

                @pl.when(jt + KB < nb)
                def _():
                    pltpu.async_copy(y_hbm.at[srcv.at[jt + KB]], gbufs[b],
                                     gsem)
            return carry

        lax.fori_loop(0, nb // KB, body, 0)
        plsc.subcore_barrier()

        @pl.when(cid == 0)
        def _():
            pltpu.sync_copy(zsh.at[pl.ds(row0, rpt)],
                            za_hbm.at[pl.ds(row0, rpt)])

        @pl.when(cid == 1)
        def _():
            pltpu.sync_copy(zsh.at[pl.ds(row0, rpt)],
                            zb_hbm.at[pl.ds(row0, rpt)])

    return prop


def _make_degree(n_pad, nb):
    rpt = n_pad // NS

    @functools.partial(
        pl.kernel,
        out_type=(jax.ShapeDtypeStruct((n_pad, DW), F32),
                  jax.ShapeDtypeStruct((n_pad, DW), F32)),
        mesh=_sc_mesh(),
        compiler_params=pltpu.CompilerParams(use_tc_tiling_on_sc=False),
        scratch_types=[
            pltpu.VMEM((nb, EB), jnp.int32),
            pltpu.VMEM((EB, DW), F32),
            pltpu.VMEM_SHARED((n_pad, DW), F32),
        ],
    )
    def degk(dst_hbm, ones_hbm, zeros_hbm, da_hbm, db_hbm, dstv, onesv, dsh):
        cid = lax.axis_index("c")
        sid = lax.axis_index("s")
        wid = sid * NC + cid
        pltpu.sync_copy(dst_hbm.at[wid], dstv)
        pltpu.sync_copy(ones_hbm, onesv)
        row0 = sid * rpt
        pltpu.sync_copy(zeros_hbm.at[pl.ds(row0, rpt)],
                        dsh.at[pl.ds(row0, rpt)])
        plsc.subcore_barrier()

        def body(j, carry):
            pltpu.sync_copy(onesv, dsh.at[dstv.at[j]], add=True)
            return carry

        lax.fori_loop(0, nb, body, 0)
        plsc.subcore_barrier()

        @pl.when(cid == 0)
        def _():
            pltpu.sync_copy(dsh.at[pl.ds(row0, rpt)],
                            da_hbm.at[pl.ds(row0, rpt)])

        @pl.when(cid == 1)
        def _():
            pltpu.sync_copy(dsh.at[pl.ds(row0, rpt)],
                            db_hbm.at[pl.ds(row0, rpt)])

    return degk


def _full(shape):
    return pl.BlockSpec(shape, lambda i: tuple(0 for _ in shape))


def _rows(c):
    return pl.BlockSpec((BM, c), lambda i: (i, 0))


def _rows_flat(c):
    return pl.BlockSpec((BM * c // 128, 128), lambda i: (i, 0))


def _inproj(x, dega, degb, win, bin_, w1, n):
    grid = (n // BM,)
    c0 = win.shape[1]
    c1 = w1.shape[1]

    def body(x_ref, da_ref, db_ref, w_ref, b_ref, w1_ref,
             h0_ref, dinv_ref, y1_ref):
        deg = da_ref[:, 0:1] + db_ref[:, 0:1] + 1.0
        dv = lax.rsqrt(deg)
        h0 = jnp.dot(x_ref[...], w_ref[...],
                     preferred_element_type=F32) + b_ref[...]
        h0_ref[...] = h0
        dinv_ref[...] = jnp.broadcast_to(dv, dinv_ref.shape)
        y1_ref[...] = dv * jnp.dot(h0, w1_ref[...], preferred_element_type=F32)

    return pl.pallas_call(
        body,
        grid=grid,
        in_specs=[_rows(x.shape[1]), _rows(DW), _rows(DW),
                  _full(win.shape), _full(bin_.shape), _full(w1.shape)],
        out_specs=(_rows(c0), _rows(8), _rows(c1)),
        out_shape=(jax.ShapeDtypeStruct((n, c0), F32),
                   jax.ShapeDtypeStruct((n, 8), F32),
                   jax.ShapeDtypeStruct((n, c1), F32)),
    )(x, dega, degb, win, bin_, w1)


def _matmul_main(feats, scshs, w, widths, n):
    grid = (n // BM,)
    nf = len(feats)
    c_out = w.shape[1]

    def body(*refs):
        f_refs = refs[0:nf]
        s_refs = refs[nf:2 * nf - 1]
        w_ref = refs[2 * nf - 1]
        y_ref = refs[2 * nf]
        off = widths[0]
        acc = jnp.dot(f_refs[0][...], w_ref[0:off, :],
                      preferred_element_type=F32)
        for j in range(1, nf):
            sc = s_refs[j - 1]
            o = jnp.maximum(f_refs[j][...] * sc[0:1, :] + sc[1:2, :], 0.0)
            acc = acc + jnp.dot(o, w_ref[off:off + widths[j], :],
                                preferred_element_type=F32)
            off += widths[j]
        y_ref[...] = acc

    in_specs = ([_rows(f.shape[1]) for f in feats]
                + [_full(s.shape) for s in scshs]
                + [_full(w.shape)])
    return pl.pallas_call(
        body,
        grid=grid,
        in_specs=in_specs,
        out_specs=_rows(c_out),
        out_shape=jax.ShapeDtypeStruct((n, c_out), F32),
    )(*feats, *scshs, w)


def _flat_ok(c):
    return c == 128


def _matmul_tail(acc, u, scsh, dinv, w, off, wd, n):
    grid = (n // BM,)
    c_out = w.shape[1]
    c = u.shape[1]

    def body(a_ref, u_ref, sc_ref, dinv_ref, w_ref, y_ref):
        o = jnp.maximum(u_ref[...] * sc_ref[0:1, :] + sc_ref[1:2, :], 0.0)
        y_ref[...] = dinv_ref[:, 0:1] * (
            a_ref[...] + jnp.dot(o, w_ref[off:off + wd, :],
                                 preferred_element_type=F32))

    return pl.pallas_call(
        body,
        grid=grid,
        in_specs=[_rows(c_out), _rows(c), _full(scsh.shape), _rows(8),
                  _full(w.shape)],
        out_specs=_rows(c_out),
        out_shape=jax.ShapeDtypeStruct((n, c_out), F32),
    )(acc, u, scsh, dinv, w)


def _fin(za, zb, y, dinv, gamma, beta, c, n):
    grid = (n // BM,)
    flat = _flat_ok(c)

    def body(za_ref, zb_ref, y_ref, dinv_ref, g_ref, b_ref, u_ref, sc_ref, acc):
        i = pl.program_id(0)

        @pl.when(i == 0)
        def _():
            acc[...] = jnp.zeros_like(acc)

        dv = dinv_ref[:, 0:1]
        z = za_ref[...] + zb_ref[...] + y_ref[...]
        u = dv * (z.reshape(BM, c) if flat else z)
        u_ref[...] = u
        acc[0:1, :] += jnp.sum(u, axis=0, keepdims=True)
        acc[1:2, :] += jnp.sum(u * u, axis=0, keepdims=True)

        @pl.when(i == grid[0] - 1)
        def _():
            mean = acc[0:1, :] / float(n)
            var = acc[1:2, :] / float(n) - mean * mean
            rstd = lax.rsqrt(var + 1e-5)
            scale = g_ref[...] * rstd
            shift = b_ref[...] - mean * scale
            sc_ref[...] = jnp.concatenate(
                [scale, shift, jnp.zeros((6, c), F32)], axis=0)

    zspec = _rows_flat(c) if flat else _rows(c)
    return pl.pallas_call(
        body,
        grid=grid,
        in_specs=[zspec, zspec, zspec, _rows(8),
                  _full((1, c)), _full((1, c))],
        out_specs=(_rows(c), _full((8, c))),
        out_shape=(jax.ShapeDtypeStruct((n, c), F32),
                   jax.ShapeDtypeStruct((8, c), F32)),
        scratch_shapes=[pltpu.VMEM((8, c), F32)],
    )(za, zb, y, dinv, gamma, beta)


def _pool_cls(u, sc, batch2, w1, b1, w2, b2, n, g):
    grid = (n // BM,)
    c = u.shape[1]
    nclass = w2.shape[1]

    def body(u_ref, sc_ref, b_ref, w1_ref, b1_ref, w2_ref, b2_ref,
             out_ref, accg, accc):
        i = pl.program_id(0)

        @pl.when(i == 0)
        def _():
            accg[...] = jnp.zeros_like(accg)
            accc[...] = jnp.zeros_like(accc)

        scale = sc_ref[0:1, :]
        shift = sc_ref[1:2, :]
        o = jnp.maximum(u_ref[...] * scale + shift, 0.0)
        gid = b_ref[...]
        onehot = (gid == lax.broadcasted_iota(jnp.int32, (1, g), 1)).astype(F32)
        dn = (((0,), (0,)), ((), ()))
        accg[...] += lax.dot_general(onehot, o, dn, preferred_element_type=F32)
        accc[...] += lax.dot_general(onehot, jnp.ones_like(o), dn,
                                     preferred_element_type=F32)

        @pl.when(i == grid[0] - 1)
        def _():
            gm = accg[...] / jnp.maximum(accc[...], 1.0)
            z1 = jnp.maximum(
                jnp.dot(gm, w1_ref[...], preferred_element_type=F32)
                + b1_ref[...], 0.0)
            out_ref[...] = (jnp.dot(z1, w2_ref[...], preferred_element_type=F32)
                            + b2_ref[...])

    return pl.pallas_call(
        body,
        grid=grid,
        in_specs=[_rows(c), _full((8, c)), _rows(1),
                  _full(w1.shape), _full(b1.shape),
                  _full(w2.shape), _full(b2.shape)],
        out_specs=_full((g, nclass)),
        out_shape=jax.ShapeDtypeStruct((g, nclass), F32),
        scratch_shapes=[pltpu.VMEM((g, c), F32), pltpu.VMEM((g, c), F32)],
    )(u, sc, batch2, w1, b1, w2, b2)


def kernel(x, edge_index, batch, params):
    n = x.shape[0]
    e = edge_index.shape[1]
    n_pad = -(-(n + DW) // 128) * 128

    src = edge_index[0].astype(jnp.int32)
    dst = edge_index[1].astype(jnp.int32)

    ep = -(-e // (NW * EB)) * (NW * EB)
    extra = ep - e
    padv = jnp.arange(extra, dtype=jnp.int32) % DW
    src3 = jnp.concatenate([src, padv]).reshape(NW, -1, EB)
    dst3 = jnp.concatenate([dst, n + padv]).reshape(NW, -1, EB)
    nb = src3.shape[1]

    zeros16 = jnp.zeros((n_pad, DW), F32)
    ones16 = jnp.ones((EB, DW), F32)

    dega, degb = _make_degree(n_pad, nb)(dst3, ones16, zeros16)

    convs = []
    for blk in params["blocks"]:
        for lp in blk:
            convs.append((lp["gcn"]["W"], lp["bn"]["gamma"], lp["bn"]["beta"]))
    convs.append((params["final_gcn"]["W"], params["final_bn"]["gamma"],
                  params["final_bn"]["beta"]))

    win = params["in_proj"]["W"]
    bin_ = params["in_proj"]["b"].reshape(1, -1)
    h0, dinv, y = _inproj(x, dega, degb, win, bin_, convs[0][0], n)

    feats = [h0]
    scshs = []
    widths = [h0.shape[1]]
    zeros_cache = {}
    for k, (w, gamma, beta) in enumerate(convs):
        c_out = w.shape[1]
        if c_out not in zeros_cache:
            zeros_cache[c_out] = jnp.zeros((n_pad, c_out), F32)
        za, zb = _make_propagate(n_pad, nb, c_out)(
            src3, dst3, y, zeros_cache[c_out])
        if k + 1 < len(convs):
            w_next = convs[k + 1][0]
            acc_next = _matmul_main(feats, scshs, w_next, widths, n)
        u, scsh = _fin(za, zb, y, dinv, gamma.reshape(1, -1),
                       beta.reshape(1, -1), c_out, n)
        feats.append(u)
        scshs.append(scsh)
        widths.append(c_out)
        if k + 1 < len(convs):
            y = _matmul_tail(acc_next, u, scsh, dinv, w_next,
                             sum(widths[:-1]), c_out, n)

    u_final = feats.pop()
    sc_final = scshs.pop()
    batch2 = batch.astype(jnp.int32).reshape(n, 1)
    g = 64
    return _pool_cls(u_final, sc_final, batch2,
                     params["cls1"]["W"], params["cls1"]["b"].reshape(1, -1),
                     params["cls2"]["W"], params["cls2"]["b"].reshape(1, -1),
                     n, g)


# speedup vs baseline: 1.1865x; 1.1398x over previous
import functools

import jax
import jax.numpy as jnp
from jax import lax
from jax.experimental import pallas as pl
from jax.experimental.pallas import tpu as pltpu
from jax.experimental.pallas import tpu_sc as plsc

F32 = jnp.float32
NC, NS = 2, 16
NW = NC * NS
EB = 128
DW = 16
BM = 2000


def _sc_mesh():
    return plsc.VectorSubcoreMesh(
        core_axis_name="c", subcore_axis_name="s", num_cores=NC, num_subcores=NS)


def _make_propagate(n_pad, nb, c):
    rpt = n_pad // NS
    KB = 2 if c >= 128 else 5
    assert nb % KB == 0

    @functools.partial(
        pl.kernel,
        out_type=(jax.ShapeDtypeStruct((n_pad, c), F32),
                  jax.ShapeDtypeStruct((n_pad, c), F32)),
        mesh=_sc_mesh(),
        compiler_params=pltpu.CompilerParams(use_tc_tiling_on_sc=False),
        scratch_types=[
            pltpu.VMEM((nb, EB), jnp.int32),
            pltpu.VMEM((nb, EB), jnp.int32),
            [pltpu.VMEM((EB, c), F32) for _ in range(KB)],
            pltpu.VMEM_SHARED((n_pad, c), F32),
            pltpu.SemaphoreType.DMA,
        ],
    )
    def prop(src_hbm, dst_hbm, y_hbm, zeros_hbm, za_hbm, zb_hbm,
             srcv, dstv, gbufs, zsh, gsem):
        cid = lax.axis_index("c")
        sid = lax.axis_index("s")
        wid = sid * NC + cid
        pltpu.sync_copy(src_hbm.at[wid], srcv)
        for b in range(KB):
            pltpu.async_copy(y_hbm.at[srcv.at[b]], gbufs[b], gsem)
        pltpu.sync_copy(dst_hbm.at[wid], dstv)
        row0 = sid * rpt
        pltpu.sync_copy(zeros_hbm.at[pl.ds(row0, rpt)],
                        zsh.at[pl.ds(row0, rpt)])
        plsc.subcore_barrier()

        def body(g, carry):
            for b in range(KB):
                jt = g * KB + b
                pltpu.make_async_copy(y_hbm.at[srcv.at[jt]], gbufs[b],
                                      gsem).wait()
                pltpu.sync_copy(gbufs[b], zsh.at[dstv.at[jt]], add=True)

                @pl.when(jt + KB < nb)
                def _():
                    pltpu.async_copy(y_hbm.at[srcv.at[jt + KB]], gbufs[b],
                                     gsem)
            return carry

        lax.fori_loop(0, nb // KB, body, 0)
        plsc.subcore_barrier()

        @pl.when(cid == 0)
        def _():
            pltpu.sync_copy(zsh.at[pl.ds(row0, rpt)],
                            za_hbm.at[pl.ds(row0, rpt)])

        @pl.when(cid == 1)
        def _():
            pltpu.sync_copy(zsh.at[pl.ds(row0, rpt)],
                            zb_hbm.at[pl.ds(row0, rpt)])

    return prop


def _make_degree(n_pad, nb):
    rpt = n_pad // NS

    @functools.partial(
        pl.kernel,
        out_type=(jax.ShapeDtypeStruct((n_pad, DW), F32),
                  jax.ShapeDtypeStruct((n_pad, DW), F32)),
        mesh=_sc_mesh(),
        compiler_params=pltpu.CompilerParams(use_tc_tiling_on_sc=False),
        scratch_types=[
            pltpu.VMEM((nb, EB), jnp.int32),
            pltpu.VMEM((EB, DW), F32),
            pltpu.VMEM_SHARED((n_pad, DW), F32),
        ],
    )
    def degk(dst_hbm, ones_hbm, zeros_hbm, da_hbm, db_hbm, dstv, onesv, dsh):
        cid = lax.axis_index("c")
        sid = lax.axis_index("s")
        wid = sid * NC + cid
        pltpu.sync_copy(dst_hbm.at[wid], dstv)
        pltpu.sync_copy(ones_hbm, onesv)
        row0 = sid * rpt
        pltpu.sync_copy(zeros_hbm.at[pl.ds(row0, rpt)],
                        dsh.at[pl.ds(row0, rpt)])
        plsc.subcore_barrier()

        def body(j, carry):
            pltpu.sync_copy(onesv, dsh.at[dstv.at[j]], add=True)
            return carry

        lax.fori_loop(0, nb, body, 0)
        plsc.subcore_barrier()

        @pl.when(cid == 0)
        def _():
            pltpu.sync_copy(dsh.at[pl.ds(row0, rpt)],
                            da_hbm.at[pl.ds(row0, rpt)])

        @pl.when(cid == 1)
        def _():
            pltpu.sync_copy(dsh.at[pl.ds(row0, rpt)],
                            db_hbm.at[pl.ds(row0, rpt)])

    return degk


def _full(shape):
    return pl.BlockSpec(shape, lambda i: tuple(0 for _ in shape))


def _rows(c):
    return pl.BlockSpec((BM, c), lambda i: (i, 0))


def _rows_flat(c):
    return pl.BlockSpec((BM * c // 128, 128), lambda i: (i, 0))


def _inproj(x, dega, degb, win, bin_, w1, n):
    grid = (n // BM,)
    c0 = win.shape[1]
    c1 = w1.shape[1]

    def body(x_ref, da_ref, db_ref, w_ref, b_ref, w1_ref,
             h0_ref, dinv_ref, y1_ref):
        deg = da_ref[:, 0:1] + db_ref[:, 0:1] + 1.0
        dv = lax.rsqrt(deg)
        h0 = jnp.dot(x_ref[...], w_ref[...],
                     preferred_element_type=F32) + b_ref[...]
        h0_ref[...] = h0
        dinv_ref[...] = jnp.broadcast_to(dv, dinv_ref.shape)
        y1 = dv * jnp.dot(h0, w1_ref[...], preferred_element_type=F32)
        if c1 < 128:
            y1 = jnp.concatenate([y1, jnp.zeros((BM, 128 - c1), F32)], axis=1)
        y1_ref[...] = y1

    return pl.pallas_call(
        body,
        grid=grid,
        in_specs=[_rows(x.shape[1]), _rows(DW), _rows(DW),
                  _full(win.shape), _full(bin_.shape), _full(w1.shape)],
        out_specs=(_rows(c0), _rows(8), _rows(128)),
        out_shape=(jax.ShapeDtypeStruct((n, c0), F32),
                   jax.ShapeDtypeStruct((n, 8), F32),
                   jax.ShapeDtypeStruct((n, 128), F32)),
    )(x, dega, degb, win, bin_, w1)


def _matmul_main(feats, scshs, w, widths, n):
    grid = (n // BM,)
    nf = len(feats)
    c_out = w.shape[1]

    def body(*refs):
        f_refs = refs[0:nf]
        s_refs = refs[nf:2 * nf - 1]
        w_ref = refs[2 * nf - 1]
        y_ref = refs[2 * nf]
        off = widths[0]
        acc = jnp.dot(f_refs[0][...], w_ref[0:off, :],
                      preferred_element_type=F32)
        for j in range(1, nf):
            sc = s_refs[j - 1]
            o = jnp.maximum(f_refs[j][...] * sc[0:1, :] + sc[1:2, :], 0.0)
            acc = acc + jnp.dot(o, w_ref[off:off + widths[j], :],
                                preferred_element_type=F32)
            off += widths[j]
        y_ref[...] = acc

    in_specs = ([_rows(f.shape[1]) for f in feats]
                + [_full(s.shape) for s in scshs]
                + [_full(w.shape)])
    return pl.pallas_call(
        body,
        grid=grid,
        in_specs=in_specs,
        out_specs=_rows(c_out),
        out_shape=jax.ShapeDtypeStruct((n, c_out), F32),
    )(*feats, *scshs, w)


def _flat_ok(c):
    return c == 128


def _matmul_tail(acc, u, scsh, dinv, w, off, wd, n):
    grid = (n // BM,)
    c_out = w.shape[1]
    c = u.shape[1]

    def body(a_ref, u_ref, sc_ref, dinv_ref, w_ref, y_ref):
        o = jnp.maximum(u_ref[...] * sc_ref[0:1, :] + sc_ref[1:2, :], 0.0)
        y = dinv_ref[:, 0:1] * (
            a_ref[...] + jnp.dot(o, w_ref[off:off + wd, :],
                                 preferred_element_type=F32))
        if c_out < 128:
            y = jnp.concatenate([y, jnp.zeros((BM, 128 - c_out), F32)], axis=1)
        y_ref[...] = y

    return pl.pallas_call(
        body,
        grid=grid,
        in_specs=[_rows(c_out), _rows(c), _full(scsh.shape), _rows(8),
                  _full(w.shape)],
        out_specs=_rows(128),
        out_shape=jax.ShapeDtypeStruct((n, 128), F32),
    )(acc, u, scsh, dinv, w)


def _fin(za, zb, y, dinv, gamma, beta, c, n):
    grid = (n // BM,)
    flat = _flat_ok(c)

    def body(za_ref, zb_ref, y_ref, dinv_ref, g_ref, b_ref, u_ref, sc_ref, acc):
        i = pl.program_id(0)

        @pl.when(i == 0)
        def _():
            acc[...] = jnp.zeros_like(acc)

        dv = dinv_ref[:, 0:1]
        z = za_ref[...] + zb_ref[...]
        u = dv * ((z.reshape(BM, c) if flat else z) + y_ref[...][:, 0:c])
        u_ref[...] = u
        acc[0:1, :] += jnp.sum(u, axis=0, keepdims=True)
        acc[1:2, :] += jnp.sum(u * u, axis=0, keepdims=True)

        @pl.when(i == grid[0] - 1)
        def _():
            mean = acc[0:1, :] / float(n)
            var = acc[1:2, :] / float(n) - mean * mean
            rstd = lax.rsqrt(var + 1e-5)
            scale = g_ref[...] * rstd
            shift = b_ref[...] - mean * scale
            sc_ref[...] = jnp.concatenate(
                [scale, shift, jnp.zeros((6, c), F32)], axis=0)

    zspec = _rows_flat(c) if flat else _rows(c)
    return pl.pallas_call(
        body,
        grid=grid,
        in_specs=[zspec, zspec, _rows(128), _rows(8),
                  _full((1, c)), _full((1, c))],
        out_specs=(_rows(c), _full((8, c))),
        out_shape=(jax.ShapeDtypeStruct((n, c), F32),
                   jax.ShapeDtypeStruct((8, c), F32)),
        scratch_shapes=[pltpu.VMEM((8, c), F32)],
    )(za, zb, y, dinv, gamma, beta)


def _pool_cls(u, sc, batch2, w1, b1, w2, b2, n, g):
    grid = (n // BM,)
    c = u.shape[1]
    nclass = w2.shape[1]

    def body(u_ref, sc_ref, b_ref, w1_ref, b1_ref, w2_ref, b2_ref,
             out_ref, accg, accc):
        i = pl.program_id(0)

        @pl.when(i == 0)
        def _():
            accg[...] = jnp.zeros_like(accg)
            accc[...] = jnp.zeros_like(accc)

        scale = sc_ref[0:1, :]
        shift = sc_ref[1:2, :]
        o = jnp.maximum(u_ref[...] * scale + shift, 0.0)
        gid = b_ref[...]
        onehot = (gid == lax.broadcasted_iota(jnp.int32, (1, g), 1)).astype(F32)
        dn = (((0,), (0,)), ((), ()))
        accg[...] += lax.dot_general(onehot, o, dn, preferred_element_type=F32)
        accc[...] += lax.dot_general(onehot, jnp.ones_like(o), dn,
                                     preferred_element_type=F32)

        @pl.when(i == grid[0] - 1)
        def _():
            gm = accg[...] / jnp.maximum(accc[...], 1.0)
            z1 = jnp.maximum(
                jnp.dot(gm, w1_ref[...], preferred_element_type=F32)
                + b1_ref[...], 0.0)
            out_ref[...] = (jnp.dot(z1, w2_ref[...], preferred_element_type=F32)
                            + b2_ref[...])

    return pl.pallas_call(
        body,
        grid=grid,
        in_specs=[_rows(c), _full((8, c)), _rows(1),
                  _full(w1.shape), _full(b1.shape),
                  _full(w2.shape), _full(b2.shape)],
        out_specs=_full((g, nclass)),
        out_shape=jax.ShapeDtypeStruct((g, nclass), F32),
        scratch_shapes=[pltpu.VMEM((g, c), F32), pltpu.VMEM((g, c), F32)],
    )(u, sc, batch2, w1, b1, w2, b2)


def kernel(x, edge_index, batch, params):
    n = x.shape[0]
    e = edge_index.shape[1]
    n_pad = -(-(n + DW) // 128) * 128

    src = edge_index[0].astype(jnp.int32)
    dst = edge_index[1].astype(jnp.int32)

    ep = -(-e // (NW * EB)) * (NW * EB)
    extra = ep - e
    padv = jnp.arange(extra, dtype=jnp.int32) % DW
    src3 = jnp.concatenate([src, padv]).reshape(NW, -1, EB)
    dst3 = jnp.concatenate([dst, n + padv]).reshape(NW, -1, EB)
    nb = src3.shape[1]
    src3_by_f = {1: src3, 2: src3 * 2, 4: src3 * 4}

    zeros16 = jnp.zeros((n_pad, DW), F32)
    ones16 = jnp.ones((EB, DW), F32)

    dega, degb = _make_degree(n_pad, nb)(dst3, ones16, zeros16)

    convs = []
    for blk in params["blocks"]:
        for lp in blk:
            convs.append((lp["gcn"]["W"], lp["bn"]["gamma"], lp["bn"]["beta"]))
    convs.append((params["final_gcn"]["W"], params["final_bn"]["gamma"],
                  params["final_bn"]["beta"]))

    win = params["in_proj"]["W"]
    bin_ = params["in_proj"]["b"].reshape(1, -1)
    h0, dinv, y = _inproj(x, dega, degb, win, bin_, convs[0][0], n)

    feats = [h0]
    scshs = []
    widths = [h0.shape[1]]
    zeros_cache = {}
    for k, (w, gamma, beta) in enumerate(convs):
        c_out = w.shape[1]
        if c_out not in zeros_cache:
            zeros_cache[c_out] = jnp.zeros((n_pad, c_out), F32)
        f = 128 // c_out
        za, zb = _make_propagate(n_pad, nb, c_out)(
            src3_by_f[f], dst3, y.reshape(n * f, c_out), zeros_cache[c_out])
        if k + 1 < len(convs):
            w_next = convs[k + 1][0]
            acc_next = _matmul_main(feats, scshs, w_next, widths, n)
        u, scsh = _fin(za, zb, y, dinv, gamma.reshape(1, -1),
                       beta.reshape(1, -1), c_out, n)
        feats.append(u)
        scshs.append(scsh)
        widths.append(c_out)
        if k + 1 < len(convs):
            y = _matmul_tail(acc_next, u, scsh, dinv, w_next,
                             sum(widths[:-1]), c_out, n)

    u_final = feats.pop()
    sc_final = scshs.pop()
    batch2 = batch.astype(jnp.int32).reshape(n, 1)
    g = 64
    return _pool_cls(u_final, sc_final, batch2,
                     params["cls1"]["W"], params["cls1"]["b"].reshape(1, -1),
                     params["cls2"]["W"], params["cls2"]["b"].reshape(1, -1),
                     n, g)
